# Initial kernel scaffold; baseline (speedup 1.0000x reference)
import jax
import jax.numpy as jnp
from jax.experimental import pallas as pl


def kernel(x, g_prompts, e_prompts, e_keys, cls_token, task_id):
    raise NotImplementedError("write your pallas kernel here")


# speedup vs baseline: 1.2008x; 1.2008x over previous
import jax
import jax.numpy as jnp
from jax.experimental import pallas as pl
from jax.experimental.pallas import tpu as pltpu

TOP_K = 5
PROMPT_LEN = 8
POOL = 100


def _body(task_ref, x_ref, g_ref, ep_ref, ek_ref, cls_ref, out_ref):
    xb = x_ref[0]
    q = jnp.mean(xb, axis=0, keepdims=True)
    qn = q / jnp.maximum(jnp.sqrt(jnp.sum(q * q)), 1e-12)
    ek = ek_ref[...]
    kn = ek / jnp.maximum(
        jnp.sqrt(jnp.sum(ek * ek, axis=1, keepdims=True)), 1e-12)
    sim = jax.lax.dot_general(
        qn, kn, (((1,), (1,)), ((), ())),
        preferred_element_type=jnp.float32)

    tid = task_ref[0]
    out_ref[0, 0:PROMPT_LEN, :] = g_ref[pl.ds(tid * PROMPT_LEN, PROMPT_LEN), :]

    col = jax.lax.broadcasted_iota(jnp.int32, (1, POOL), 1)
    for k in range(TOP_K):
        idx = jnp.argmax(sim[0])
        rows = ep_ref[pl.ds(idx * PROMPT_LEN, PROMPT_LEN), :]
        base = PROMPT_LEN + k * PROMPT_LEN
        out_ref[0, base:base + PROMPT_LEN, :] = rows
        sim = jnp.where(col == idx, -jnp.inf, sim)

    ccol = (TOP_K + 1) * PROMPT_LEN
    out_ref[0, ccol:ccol + 1, :] = cls_ref[...]
    out_ref[0, ccol + 1:, :] = xb


def kernel(x, g_prompts, e_prompts, e_keys, cls_token, task_id):
    B, S, d = x.shape
    n_out = (TOP_K + 1) * PROMPT_LEN + 1 + S
    g_flat = g_prompts.reshape(-1, d)
    ep_flat = e_prompts.reshape(-1, d)
    cls2 = cls_token.reshape(1, d)
    task = jnp.asarray(task_id, jnp.int32).reshape(1)
    return pl.pallas_call(
        _body,
        grid=(B,),
        in_specs=[
            pl.BlockSpec(memory_space=pltpu.SMEM),
            pl.BlockSpec((1, S, d), lambda b: (b, 0, 0)),
            pl.BlockSpec(g_flat.shape, lambda b: (0, 0)),
            pl.BlockSpec(ep_flat.shape, lambda b: (0, 0)),
            pl.BlockSpec(e_keys.shape, lambda b: (0, 0)),
            pl.BlockSpec(cls2.shape, lambda b: (0, 0)),
        ],
        out_specs=pl.BlockSpec((1, n_out, d), lambda b: (b, 0, 0)),
        out_shape=jax.ShapeDtypeStruct((B, n_out, d), x.dtype),
    )(task, x, g_flat, ep_flat, e_keys, cls2)
